# baseline (speedup 1.0000x reference)
import jax
import jax.numpy as jnp
from jax.experimental import pallas as pl
from jax.experimental.pallas import tpu as pltpu

N_CHUNKS = 8


def _copy_body(u_in, i_in, u_out, i_out, sem_u, sem_i):
    n = u_in.shape[0]
    chunk = n // N_CHUNKS
    copies = []
    for c in range(N_CHUNKS):
        s = pl.ds(c * chunk, chunk)
        cu = pltpu.make_async_copy(u_in.at[s], u_out.at[s], sem_u.at[c])
        ci = pltpu.make_async_copy(i_in.at[s], i_out.at[s], sem_i.at[c])
        cu.start()
        ci.start()
        copies.append((cu, ci))
    for cu, ci in copies:
        cu.wait()
        ci.wait()


def kernel(user_emb, item_emb):
    out_shapes = (
        jax.ShapeDtypeStruct(user_emb.shape, user_emb.dtype),
        jax.ShapeDtypeStruct(item_emb.shape, item_emb.dtype),
    )
    return pl.pallas_call(
        _copy_body,
        out_shape=out_shapes,
        in_specs=[
            pl.BlockSpec(memory_space=pl.ANY),
            pl.BlockSpec(memory_space=pl.ANY),
        ],
        out_specs=[
            pl.BlockSpec(memory_space=pl.ANY),
            pl.BlockSpec(memory_space=pl.ANY),
        ],
        scratch_shapes=[
            pltpu.SemaphoreType.DMA((N_CHUNKS,)),
            pltpu.SemaphoreType.DMA((N_CHUNKS,)),
        ],
    )(user_emb, item_emb)


# speedup vs baseline: 18.1371x; 18.1371x over previous
import jax
import jax.numpy as jnp
from jax.experimental import pallas as pl
from jax.experimental.pallas import tpu as pltpu

BLOCK_ROWS = 8000


def _copy_body(u_in, i_in, u_out, i_out):
    u_out[...] = u_in[...]
    i_out[...] = i_in[...]


def kernel(user_emb, item_emb):
    n, d = user_emb.shape
    grid = (n // BLOCK_ROWS,)
    spec = pl.BlockSpec((BLOCK_ROWS, d), lambda g: (g, 0))
    return pl.pallas_call(
        _copy_body,
        grid=grid,
        out_shape=(
            jax.ShapeDtypeStruct(user_emb.shape, user_emb.dtype),
            jax.ShapeDtypeStruct(item_emb.shape, item_emb.dtype),
        ),
        in_specs=[spec, spec],
        out_specs=[spec, spec],
    )(user_emb, item_emb)


# speedup vs baseline: 203.0182x; 11.1935x over previous
import jax
import jax.numpy as jnp
from jax.experimental import pallas as pl
from jax.experimental.pallas import tpu as pltpu

BLOCK_COLS = 32768


def _copy_body(u_in, i_in, u_out, i_out):
    u_out[...] = u_in[...]
    i_out[...] = i_in[...]


def kernel(user_emb, item_emb):
    ut = user_emb.T
    it = item_emb.T
    d, n = ut.shape
    grid = (pl.cdiv(n, BLOCK_COLS),)
    spec = pl.BlockSpec((d, BLOCK_COLS), lambda g: (0, g))
    out_ut, out_it = pl.pallas_call(
        _copy_body,
        grid=grid,
        out_shape=(
            jax.ShapeDtypeStruct(ut.shape, ut.dtype),
            jax.ShapeDtypeStruct(it.shape, it.dtype),
        ),
        in_specs=[spec, spec],
        out_specs=[spec, spec],
    )(ut, it)
    return out_ut.T, out_it.T
